# baseline (speedup 1.0000x reference)
import functools

import jax
import jax.numpy as jnp
from jax import lax
from jax.experimental import pallas as pl
from jax.experimental.pallas import tpu as pltpu
from jax.experimental.pallas import tpu_sc as plsc

N_USERS = 30000
N_ITEMS = 70000
N = N_USERS + N_ITEMS
EMB = 32
HALF = 16
NNZ = 1600000
N_LAYERS = 3
BATCH = 4096

NC = 2
NS = 16
LANES = 16

N_PAD = 102400
K = 128
CPB = 10
BLK = CPB * K
EPT = 79 * BLK
NNZ_PAD = EPT * NS
NBLK = EPT // BLK
NPAIR = (NBLK - 1) // 2
NBUF = 5
SBUF = 4
GFULL = K // LANES
GREM = K - GFULL * LANES
CROWS = NNZ_PAD // K
RPT = N_PAD // NS
ZROWS = 64

_mesh = plsc.VectorSubcoreMesh(
    core_axis_name="c", subcore_axis_name="s", num_cores=NC, num_subcores=NS)


def _zero_f32(buf, nrows):
    zero = jnp.zeros((LANES,), jnp.float32)

    def body(r, _):
        buf[r, :] = zero
        return 0

    lax.fori_loop(0, nrows, body, 0, unroll=8)


@functools.partial(
    pl.kernel,
    out_type=(
        jax.ShapeDtypeStruct((N_PAD, HALF), jnp.float32),
        jax.ShapeDtypeStruct((N_PAD, HALF), jnp.float32),
    ),
    mesh=_mesh,
    scratch_types=[
        pltpu.VMEM_SHARED((N_PAD, HALF), jnp.float32),
        pltpu.VMEM((2, CPB, K), jnp.int32),
        pltpu.VMEM((2, CPB, K), jnp.int32),
        pltpu.VMEM((2, BLK + LANES), jnp.float32),
        pltpu.VMEM((NBUF, K, HALF), jnp.float32),
        pltpu.VMEM((SBUF, K, HALF), jnp.float32),
        pltpu.VMEM((ZROWS, HALF), jnp.float32),
        pltpu.SemaphoreType.DMA,
        pltpu.SemaphoreType.DMA,
        pltpu.SemaphoreType.DMA,
        pltpu.SemaphoreType.DMA,
        pltpu.SemaphoreType.DMA,
        pltpu.SemaphoreType.DMA,
        pltpu.SemaphoreType.DMA,
        pltpu.SemaphoreType.DMA,
        pltpu.SemaphoreType.DMA,
        pltpu.SemaphoreType.DMA,
        pltpu.SemaphoreType.DMA,
    ],
    compiler_params=pltpu.CompilerParams(use_tc_tiling_on_sc=False),
)
def _spmm(x0_hbm, x1_hbm, row_hbm, col_hbm, val_hbm, y0_hbm, y1_hbm,
          acc, colblk, rowblk, valblk, gbuf, sbuf, zbuf,
          si0, si1, sg0, sg1, sg2, sg3, sg4, ss0, ss1, ss2, ss3):
    cid = lax.axis_index("c")
    sid = lax.axis_index("s")
    sem_i = (si0, si1)
    sem_g = (sg0, sg1, sg2, sg3, sg4)
    sem_s = (ss0, ss1, ss2, ss3)

    _zero_f32(zbuf, ZROWS)
    for j in range(RPT // ZROWS):
        pltpu.sync_copy(zbuf, acc.at[pl.ds(sid * RPT + j * ZROWS, ZROWS)])
    plsc.subcore_barrier()

    def edge_loop(x_hbm):
        vbase0 = sid * EPT
        crow0 = sid * (EPT // K)

        def start_idx(kblk, bi):
            crow = crow0 + kblk * CPB
            pltpu.async_copy(col_hbm.at[pl.ds(crow, CPB)],
                             colblk.at[bi], sem_i[bi])
            pltpu.async_copy(row_hbm.at[pl.ds(crow, CPB)],
                             rowblk.at[bi], sem_i[bi])
            pltpu.async_copy(val_hbm.at[pl.ds(vbase0 + kblk * BLK, BLK)],
                             valblk.at[bi, pl.ds(0, BLK)], sem_i[bi])

        def wait_idx(bi):
            pltpu.make_async_copy(col_hbm.at[pl.ds(0, CPB)],
                                  colblk.at[bi], sem_i[bi]).wait()
            pltpu.make_async_copy(row_hbm.at[pl.ds(0, CPB)],
                                  rowblk.at[bi], sem_i[bi]).wait()
            pltpu.make_async_copy(val_hbm.at[pl.ds(0, BLK)],
                                  valblk.at[bi, pl.ds(0, BLK)],
                                  sem_i[bi]).wait()

        def start_gather(bi, c, p):
            pltpu.async_copy(x_hbm.at[colblk.at[bi].at[c]],
                             gbuf.at[p], sem_g[p])

        def wait_gather(p):
            pltpu.make_async_copy(x_hbm.at[pl.ds(0, K)],
                                  gbuf.at[p], sem_g[p]).wait()

        def start_scatter(bi, c, p):
            pltpu.async_copy(sbuf.at[p], acc.at[rowblk.at[bi].at[c]],
                             sem_s[p], add=True)

        def wait_scatter(p):
            pltpu.make_async_copy(x_hbm.at[pl.ds(0, K)],
                                  sbuf.at[p], sem_s[p]).wait()

        dnums = lax.GatherDimensionNumbers(
            offset_dims=(), collapsed_slice_dims=(0,), start_index_map=(0,))

        def compute_chunk(bi, c, p, ps):
            gb = gbuf.at[p]
            sb = sbuf.at[ps]
            vbase = c * K

            def edge16(e0, vv, e):
                bidx = jnp.full((LANES, 1), e, jnp.int32)
                vs = lax.gather(
                    vv, bidx, dnums, (1,),
                    mode=lax.GatherScatterMode.PROMISE_IN_BOUNDS)
                sb[e0 + e, :] = gb[e0 + e, :] * vs

            def group_body(g, _):
                e0 = g * LANES
                vv = valblk[bi, pl.ds(vbase + e0, LANES)]
                for e in range(LANES):
                    edge16(e0, vv, e)
                return 0

            lax.fori_loop(0, GFULL, group_body, 0)
            if GREM:
                e0 = GFULL * LANES
                vv = valblk[bi, pl.ds(vbase + e0, LANES)]
                for e in range(GREM):
                    edge16(e0, vv, e)

        def do_block(b, k, first_pred, next_pred):
            LA = NBUF - 1

            def guarded(fn):
                if next_pred is True:
                    fn()
                elif next_pred is False:
                    pass
                else:
                    pl.when(next_pred)(fn)

            for c in range(CPB):
                p = c % NBUF
                ps = (2 * b + c) % SBUF
                wait_gather(p)
                pn = (c + LA) % NBUF
                if c < CPB - LA:
                    start_gather(b, c + LA, pn)
                else:
                    nb = 1 - b

                    def nxt(c=c, pn=pn, nb=nb):
                        if c == CPB - LA:
                            wait_idx(nb)
                        start_gather(nb, c - (CPB - LA), pn)
                    guarded(nxt)
                if first_pred is not None and c < SBUF:
                    pl.when(first_pred)(lambda: wait_scatter(ps))
                else:
                    wait_scatter(ps)
                if c == 4:
                    guarded(lambda: start_idx(k + 1, 1 - b))
                compute_chunk(b, c, p, ps)
                start_scatter(b, c, ps)

        start_idx(0, 0)
        wait_idx(0)
        for c in range(NBUF - 1):
            start_gather(0, c, c)

        def blk_pair(i, _):
            do_block(0, 2 * i, i > 0, True)
            do_block(1, 2 * i + 1, None, True)
            return 0

        lax.fori_loop(0, NPAIR, blk_pair, 0)
        do_block(0, NBLK - 1, None, False)
        for p in range(SBUF):
            wait_scatter(p)

    @pl.when(cid == 0)
    def _():
        edge_loop(x0_hbm)

    @pl.when(cid == 1)
    def _():
        edge_loop(x1_hbm)

    plsc.subcore_barrier()

    @pl.when(cid == 0)
    def _():
        pltpu.sync_copy(acc.at[pl.ds(sid * RPT, RPT)],
                        y0_hbm.at[pl.ds(sid * RPT, RPT)])

    @pl.when(cid == 1)
    def _():
        pltpu.sync_copy(acc.at[pl.ds(sid * RPT, RPT)],
                        y1_hbm.at[pl.ds(sid * RPT, RPT)])


BPT = BATCH // NS
GCH = 128


@functools.partial(
    pl.kernel,
    out_type=(
        jax.ShapeDtypeStruct((N_LAYERS + 2, NC, BATCH, HALF), jnp.float32),
        jax.ShapeDtypeStruct((N_LAYERS + 2, NC, BATCH, HALF), jnp.float32),
    ),
    mesh=_mesh,
    scratch_types=[
        pltpu.VMEM((GCH,), jnp.int32),
        pltpu.VMEM((GCH, HALF), jnp.float32),
        pltpu.VMEM((GCH, HALF), jnp.float32),
        pltpu.SemaphoreType.DMA,
    ],
    compiler_params=pltpu.CompilerParams(use_tc_tiling_on_sc=False),
)
def _batch_gather(uid_hbm, iid_hbm,
                  t00, t01, t10, t11, t20, t21, t30, t31,
                  u_out, i_out, idxbuf, gbuf, accbuf, sem):
    cid = lax.axis_index("c")
    sid = lax.axis_index("s")
    base0 = sid * BPT

    def acc_add(first):
        def body(r, _):
            if first:
                accbuf[r, :] = gbuf[r, :]
            else:
                accbuf[r, :] = accbuf[r, :] + gbuf[r, :]
            return 0
        lax.fori_loop(0, GCH, body, 0, unroll=8)

    def acc_scale():
        def body(r, _):
            accbuf[r, :] = accbuf[r, :] * jnp.float32(0.25)
            return 0
        lax.fori_loop(0, GCH, body, 0, unroll=8)

    def one_half(tabs, id_hbm, out_hbm, offset):
        for h in range(BPT // GCH):
            base = base0 + h * GCH
            pltpu.sync_copy(id_hbm.at[pl.ds(base, GCH)], idxbuf)
            if offset:
                for j in range(GCH // LANES):
                    sl = pl.ds(j * LANES, LANES)
                    idxbuf[sl] = idxbuf[sl] + jnp.int32(offset)
            for l, tab in enumerate(tabs):
                pltpu.async_copy(tab.at[idxbuf], gbuf, sem).wait()
                pltpu.sync_copy(gbuf, out_hbm.at[l, cid, pl.ds(base, GCH)])
                acc_add(first=(l == 0))
            acc_scale()
            pltpu.sync_copy(accbuf, out_hbm.at[N_LAYERS + 1, cid,
                                               pl.ds(base, GCH)])

    @pl.when(cid == 0)
    def _():
        one_half((t00, t10, t20, t30), uid_hbm, u_out, 0)
        one_half((t00, t10, t20, t30), iid_hbm, i_out, N_USERS)

    @pl.when(cid == 1)
    def _():
        one_half((t01, t11, t21, t31), uid_hbm, u_out, 0)
        one_half((t01, t11, t21, t31), iid_hbm, i_out, N_USERS)


def kernel(user_id, item_id, adj_row, adj_col, adj_val, user_emb, item_emb):
    user_id = user_id.astype(jnp.int32)
    item_id = item_id.astype(jnp.int32)
    adj_row = adj_row.astype(jnp.int32)
    adj_col = adj_col.astype(jnp.int32)

    pad = jnp.zeros((N_PAD - N, HALF), jnp.float32)
    ego0 = jnp.concatenate([user_emb[:, :HALF], item_emb[:, :HALF], pad],
                           axis=0)
    ego1 = jnp.concatenate([user_emb[:, HALF:], item_emb[:, HALF:], pad],
                           axis=0)

    epad = NNZ_PAD - NNZ
    row2 = jnp.concatenate(
        [adj_row, jnp.zeros((epad,), jnp.int32)]).reshape(CROWS, K)
    col2 = jnp.concatenate(
        [adj_col, jnp.zeros((epad,), jnp.int32)]).reshape(CROWS, K)
    adj_val = jnp.concatenate([adj_val, jnp.zeros((epad,), jnp.float32)])

    halves = [(ego0, ego1)]
    for _ in range(N_LAYERS):
        x0, x1 = halves[-1]
        halves.append(_spmm(x0, x1, row2, col2, adj_val))

    tabs = [t for pair in halves for t in pair]
    u5, i5 = _batch_gather(user_id, item_id, *tabs)
    u = u5.transpose(0, 2, 1, 3).reshape(N_LAYERS + 2, BATCH, EMB)
    i = i5.transpose(0, 2, 1, 3).reshape(N_LAYERS + 2, BATCH, EMB)
    return (u, i)


# speedup vs baseline: 1.2076x; 1.2076x over previous
import functools

import jax
import jax.numpy as jnp
from jax import lax
from jax.experimental import pallas as pl
from jax.experimental.pallas import tpu as pltpu
from jax.experimental.pallas import tpu_sc as plsc

N_USERS = 30000
N_ITEMS = 70000
N = N_USERS + N_ITEMS
EMB = 32
HALF = 16
NNZ = 1600000
N_LAYERS = 3
BATCH = 4096

NC = 2
NS = 16
LANES = 16

N_PAD = 102400
EPT = NNZ // NS
K = 100
CPB = 20
BLK = CPB * K
NBLK = EPT // BLK
NPAIR = NBLK // 2
NBUF = 5
SBUF = 4
GFULL = K // LANES
GREM = K - GFULL * LANES
CROWS = NNZ // K
RPT = N_PAD // NS
ZROWS = 64

_mesh = plsc.VectorSubcoreMesh(
    core_axis_name="c", subcore_axis_name="s", num_cores=NC, num_subcores=NS)


def _zero_f32(buf, nrows):
    zero = jnp.zeros((LANES,), jnp.float32)

    def body(r, _):
        buf[r, :] = zero
        return 0

    lax.fori_loop(0, nrows, body, 0, unroll=8)


@functools.partial(
    pl.kernel,
    out_type=(
        jax.ShapeDtypeStruct((N_PAD, HALF), jnp.float32),
        jax.ShapeDtypeStruct((N_PAD, HALF), jnp.float32),
    ),
    mesh=_mesh,
    scratch_types=[
        pltpu.VMEM_SHARED((N_PAD, HALF), jnp.float32),
        pltpu.VMEM((2, CPB, K), jnp.int32),
        pltpu.VMEM((2, CPB, K), jnp.int32),
        pltpu.VMEM((2, BLK + LANES), jnp.float32),
        pltpu.VMEM((NBUF, K, HALF), jnp.float32),
        pltpu.VMEM((SBUF, K, HALF), jnp.float32),
        pltpu.VMEM((ZROWS, HALF), jnp.float32),
        pltpu.SemaphoreType.DMA,
        pltpu.SemaphoreType.DMA,
        pltpu.SemaphoreType.DMA,
        pltpu.SemaphoreType.DMA,
        pltpu.SemaphoreType.DMA,
        pltpu.SemaphoreType.DMA,
        pltpu.SemaphoreType.DMA,
        pltpu.SemaphoreType.DMA,
        pltpu.SemaphoreType.DMA,
        pltpu.SemaphoreType.DMA,
        pltpu.SemaphoreType.DMA,
    ],
    compiler_params=pltpu.CompilerParams(use_tc_tiling_on_sc=False),
)
def _spmm(x0_hbm, x1_hbm, row_hbm, col_hbm, val_hbm, y0_hbm, y1_hbm,
          acc, colblk, rowblk, valblk, gbuf, sbuf, zbuf,
          si0, si1, sg0, sg1, sg2, sg3, sg4, ss0, ss1, ss2, ss3):
    cid = lax.axis_index("c")
    sid = lax.axis_index("s")
    sem_i = (si0, si1)
    sem_g = (sg0, sg1, sg2, sg3, sg4)
    sem_s = (ss0, ss1, ss2, ss3)

    _zero_f32(zbuf, ZROWS)
    for j in range(RPT // ZROWS):
        pltpu.sync_copy(zbuf, acc.at[pl.ds(sid * RPT + j * ZROWS, ZROWS)])
    plsc.subcore_barrier()

    def edge_loop(x_hbm):
        vbase0 = sid * EPT
        crow0 = sid * (EPT // K)

        def start_idx(kblk, bi):
            crow = crow0 + kblk * CPB
            pltpu.async_copy(col_hbm.at[pl.ds(crow, CPB)],
                             colblk.at[bi], sem_i[bi])
            pltpu.async_copy(row_hbm.at[pl.ds(crow, CPB)],
                             rowblk.at[bi], sem_i[bi])
            pltpu.async_copy(val_hbm.at[pl.ds(vbase0 + kblk * BLK, BLK)],
                             valblk.at[bi, pl.ds(0, BLK)], sem_i[bi])

        def wait_idx(bi):
            pltpu.make_async_copy(col_hbm.at[pl.ds(0, CPB)],
                                  colblk.at[bi], sem_i[bi]).wait()
            pltpu.make_async_copy(row_hbm.at[pl.ds(0, CPB)],
                                  rowblk.at[bi], sem_i[bi]).wait()
            pltpu.make_async_copy(val_hbm.at[pl.ds(0, BLK)],
                                  valblk.at[bi, pl.ds(0, BLK)],
                                  sem_i[bi]).wait()

        def start_gather(bi, c, p):
            pltpu.async_copy(x_hbm.at[colblk.at[bi].at[c]],
                             gbuf.at[p], sem_g[p])

        def wait_gather(p):
            pltpu.make_async_copy(x_hbm.at[pl.ds(0, K)],
                                  gbuf.at[p], sem_g[p]).wait()

        def start_scatter(bi, c, p):
            pltpu.async_copy(sbuf.at[p], acc.at[rowblk.at[bi].at[c]],
                             sem_s[p], add=True)

        def wait_scatter(p):
            pltpu.make_async_copy(x_hbm.at[pl.ds(0, K)],
                                  sbuf.at[p], sem_s[p]).wait()

        dnums = lax.GatherDimensionNumbers(
            offset_dims=(), collapsed_slice_dims=(0,), start_index_map=(0,))

        def compute_chunk(bi, c, p, ps):
            gb = gbuf.at[p]
            sb = sbuf.at[ps]
            vbase = c * K

            def edge16(e0, vv, e):
                bidx = jnp.full((LANES, 1), e, jnp.int32)
                vs = lax.gather(
                    vv, bidx, dnums, (1,),
                    mode=lax.GatherScatterMode.PROMISE_IN_BOUNDS)
                sb[e0 + e, :] = gb[e0 + e, :] * vs

            def group_body(g, _):
                e0 = g * LANES
                vv = valblk[bi, pl.ds(vbase + e0, LANES)]
                for e in range(LANES):
                    edge16(e0, vv, e)
                return 0

            lax.fori_loop(0, GFULL, group_body, 0)
            if GREM:
                e0 = GFULL * LANES
                vv = valblk[bi, pl.ds(vbase + e0, LANES)]
                for e in range(GREM):
                    edge16(e0, vv, e)

        def do_block(b, k, first_pred, next_pred):
            LA = NBUF - 1

            def guarded(fn):
                if next_pred is True:
                    fn()
                else:
                    pl.when(next_pred)(fn)

            for c in range(CPB):
                p = c % NBUF
                ps = (2 * b + c) % SBUF
                wait_gather(p)
                pn = (c + LA) % NBUF
                if c < CPB - LA:
                    start_gather(b, c + LA, pn)
                else:
                    nb = 1 - b

                    def nxt(c=c, pn=pn, nb=nb):
                        if c == CPB - LA:
                            wait_idx(nb)
                        start_gather(nb, c - (CPB - LA), pn)
                    guarded(nxt)
                if first_pred is not None and c < SBUF:
                    pl.when(first_pred)(lambda: wait_scatter(ps))
                else:
                    wait_scatter(ps)
                if c == 4:
                    guarded(lambda: start_idx(k + 1, 1 - b))
                compute_chunk(b, c, p, ps)
                start_scatter(b, c, ps)

        start_idx(0, 0)
        wait_idx(0)
        for c in range(NBUF - 1):
            start_gather(0, c, c)

        def blk_pair(i, _):
            do_block(0, 2 * i, i > 0, True)
            do_block(1, 2 * i + 1, None, i < NPAIR - 1)
            return 0

        lax.fori_loop(0, NPAIR, blk_pair, 0)
        for p in range(SBUF):
            wait_scatter(p)

    @pl.when(cid == 0)
    def _():
        edge_loop(x0_hbm)

    @pl.when(cid == 1)
    def _():
        edge_loop(x1_hbm)

    plsc.subcore_barrier()

    @pl.when(cid == 0)
    def _():
        pltpu.sync_copy(acc.at[pl.ds(sid * RPT, RPT)],
                        y0_hbm.at[pl.ds(sid * RPT, RPT)])

    @pl.when(cid == 1)
    def _():
        pltpu.sync_copy(acc.at[pl.ds(sid * RPT, RPT)],
                        y1_hbm.at[pl.ds(sid * RPT, RPT)])


BPT = BATCH // NS
GCH = 128


@functools.partial(
    pl.kernel,
    out_type=(
        jax.ShapeDtypeStruct((N_LAYERS + 2, NC, BATCH, HALF), jnp.float32),
        jax.ShapeDtypeStruct((N_LAYERS + 2, NC, BATCH, HALF), jnp.float32),
    ),
    mesh=_mesh,
    scratch_types=[
        pltpu.VMEM((GCH,), jnp.int32),
        pltpu.VMEM((GCH, HALF), jnp.float32),
        pltpu.VMEM((GCH, HALF), jnp.float32),
        pltpu.SemaphoreType.DMA,
    ],
    compiler_params=pltpu.CompilerParams(use_tc_tiling_on_sc=False),
)
def _batch_gather(uid_hbm, iid_hbm,
                  t00, t01, t10, t11, t20, t21, t30, t31,
                  u_out, i_out, idxbuf, gbuf, accbuf, sem):
    cid = lax.axis_index("c")
    sid = lax.axis_index("s")
    base0 = sid * BPT

    def acc_add(first):
        def body(r, _):
            if first:
                accbuf[r, :] = gbuf[r, :]
            else:
                accbuf[r, :] = accbuf[r, :] + gbuf[r, :]
            return 0
        lax.fori_loop(0, GCH, body, 0, unroll=8)

    def acc_scale():
        def body(r, _):
            accbuf[r, :] = accbuf[r, :] * jnp.float32(0.25)
            return 0
        lax.fori_loop(0, GCH, body, 0, unroll=8)

    def one_half(tabs, id_hbm, out_hbm, offset):
        for h in range(BPT // GCH):
            base = base0 + h * GCH
            pltpu.sync_copy(id_hbm.at[pl.ds(base, GCH)], idxbuf)
            if offset:
                for j in range(GCH // LANES):
                    sl = pl.ds(j * LANES, LANES)
                    idxbuf[sl] = idxbuf[sl] + jnp.int32(offset)
            for l, tab in enumerate(tabs):
                pltpu.async_copy(tab.at[idxbuf], gbuf, sem).wait()
                pltpu.sync_copy(gbuf, out_hbm.at[l, cid, pl.ds(base, GCH)])
                acc_add(first=(l == 0))
            acc_scale()
            pltpu.sync_copy(accbuf, out_hbm.at[N_LAYERS + 1, cid,
                                               pl.ds(base, GCH)])

    @pl.when(cid == 0)
    def _():
        one_half((t00, t10, t20, t30), uid_hbm, u_out, 0)
        one_half((t00, t10, t20, t30), iid_hbm, i_out, N_USERS)

    @pl.when(cid == 1)
    def _():
        one_half((t01, t11, t21, t31), uid_hbm, u_out, 0)
        one_half((t01, t11, t21, t31), iid_hbm, i_out, N_USERS)


def kernel(user_id, item_id, adj_row, adj_col, adj_val, user_emb, item_emb):
    user_id = user_id.astype(jnp.int32)
    item_id = item_id.astype(jnp.int32)
    adj_row = adj_row.astype(jnp.int32)
    adj_col = adj_col.astype(jnp.int32)

    pad = jnp.zeros((N_PAD - N, HALF), jnp.float32)
    ego0 = jnp.concatenate([user_emb[:, :HALF], item_emb[:, :HALF], pad],
                           axis=0)
    ego1 = jnp.concatenate([user_emb[:, HALF:], item_emb[:, HALF:], pad],
                           axis=0)

    row2 = adj_row.reshape(CROWS, K)
    col2 = adj_col.reshape(CROWS, K)

    halves = [(ego0, ego1)]
    for _ in range(N_LAYERS):
        x0, x1 = halves[-1]
        halves.append(_spmm(x0, x1, row2, col2, adj_val))

    tabs = [t for pair in halves for t in pair]
    u5, i5 = _batch_gather(user_id, item_id, *tabs)
    u = u5.transpose(0, 2, 1, 3).reshape(N_LAYERS + 2, BATCH, EMB)
    i = i5.transpose(0, 2, 1, 3).reshape(N_LAYERS + 2, BATCH, EMB)
    return (u, i)


# speedup vs baseline: 1.2417x; 1.0282x over previous
import functools

import jax
import jax.numpy as jnp
from jax import lax
from jax.experimental import pallas as pl
from jax.experimental.pallas import tpu as pltpu
from jax.experimental.pallas import tpu_sc as plsc

N_USERS = 30000
N_ITEMS = 70000
N = N_USERS + N_ITEMS
EMB = 32
HALF = 16
NNZ = 1600000
N_LAYERS = 3
BATCH = 4096

NC = 2
NS = 16
LANES = 16

N_PAD = 102400
EPT = NNZ // NS
K = 100
CPB = 20
BLK = CPB * K
NBLK = EPT // BLK
NPAIR = NBLK // 2
NBUF = 5
SBUF = 4
GFULL = K // LANES
GREM = K - GFULL * LANES
CROWS = NNZ // K
RPT = N_PAD // NS
ZROWS = 64

_mesh = plsc.VectorSubcoreMesh(
    core_axis_name="c", subcore_axis_name="s", num_cores=NC, num_subcores=NS)


def _zero_f32(buf, nrows):
    zero = jnp.zeros((LANES,), jnp.float32)

    def body(r, _):
        buf[r, :] = zero
        return 0

    lax.fori_loop(0, nrows, body, 0, unroll=8)


@functools.partial(
    pl.kernel,
    out_type=(
        jax.ShapeDtypeStruct((N_PAD, HALF), jnp.float32),
        jax.ShapeDtypeStruct((N_PAD, HALF), jnp.float32),
    ),
    mesh=_mesh,
    scratch_types=[
        pltpu.VMEM_SHARED((N_PAD, HALF), jnp.float32),
        pltpu.VMEM((2, CPB, K), jnp.int32),
        pltpu.VMEM((2, CPB, K), jnp.int32),
        pltpu.VMEM((2, BLK + LANES), jnp.float32),
        pltpu.VMEM((NBUF, K, HALF), jnp.float32),
        pltpu.VMEM((SBUF, K, HALF), jnp.float32),
        pltpu.VMEM((ZROWS, HALF), jnp.float32),
        pltpu.SemaphoreType.DMA,
        pltpu.SemaphoreType.DMA,
        pltpu.SemaphoreType.DMA,
        pltpu.SemaphoreType.DMA,
        pltpu.SemaphoreType.DMA,
        pltpu.SemaphoreType.DMA,
        pltpu.SemaphoreType.DMA,
        pltpu.SemaphoreType.DMA,
        pltpu.SemaphoreType.DMA,
        pltpu.SemaphoreType.DMA,
        pltpu.SemaphoreType.DMA,
    ],
    compiler_params=pltpu.CompilerParams(use_tc_tiling_on_sc=False),
)
def _spmm(x0_hbm, x1_hbm, row_hbm, col_hbm, val_hbm, y0_hbm, y1_hbm,
          acc, colblk, rowblk, valblk, gbuf, sbuf, zbuf,
          si0, si1, sg0, sg1, sg2, sg3, sg4, ss0, ss1, ss2, ss3):
    cid = lax.axis_index("c")
    sid = lax.axis_index("s")
    sem_i = (si0, si1)
    sem_g = (sg0, sg1, sg2, sg3, sg4)
    sem_s = (ss0, ss1, ss2, ss3)

    _zero_f32(zbuf, ZROWS)
    for j in range(RPT // ZROWS):
        pltpu.async_copy(zbuf, acc.at[pl.ds(sid * RPT + j * ZROWS, ZROWS)],
                         si1)
    for j in range(RPT // ZROWS):
        pltpu.make_async_copy(zbuf, acc.at[pl.ds(sid * RPT, ZROWS)],
                              si1).wait()
    plsc.subcore_barrier()

    def edge_loop(x_hbm):
        vbase0 = sid * EPT
        crow0 = sid * (EPT // K)

        def start_idx(kblk, bi):
            crow = crow0 + kblk * CPB
            pltpu.async_copy(col_hbm.at[pl.ds(crow, CPB)],
                             colblk.at[bi], sem_i[bi])
            pltpu.async_copy(row_hbm.at[pl.ds(crow, CPB)],
                             rowblk.at[bi], sem_i[bi])
            pltpu.async_copy(val_hbm.at[pl.ds(vbase0 + kblk * BLK, BLK)],
                             valblk.at[bi, pl.ds(0, BLK)], sem_i[bi])

        def wait_idx(bi):
            pltpu.make_async_copy(col_hbm.at[pl.ds(0, CPB)],
                                  colblk.at[bi], sem_i[bi]).wait()
            pltpu.make_async_copy(row_hbm.at[pl.ds(0, CPB)],
                                  rowblk.at[bi], sem_i[bi]).wait()
            pltpu.make_async_copy(val_hbm.at[pl.ds(0, BLK)],
                                  valblk.at[bi, pl.ds(0, BLK)],
                                  sem_i[bi]).wait()

        def start_gather(bi, c, p):
            pltpu.async_copy(x_hbm.at[colblk.at[bi].at[c]],
                             gbuf.at[p], sem_g[p])

        def wait_gather(p):
            pltpu.make_async_copy(x_hbm.at[pl.ds(0, K)],
                                  gbuf.at[p], sem_g[p]).wait()

        def start_scatter(bi, c, p):
            pltpu.async_copy(sbuf.at[p], acc.at[rowblk.at[bi].at[c]],
                             sem_s[p], add=True)

        def wait_scatter(p):
            pltpu.make_async_copy(x_hbm.at[pl.ds(0, K)],
                                  sbuf.at[p], sem_s[p]).wait()

        dnums = lax.GatherDimensionNumbers(
            offset_dims=(), collapsed_slice_dims=(0,), start_index_map=(0,))

        def compute_chunk(bi, c, p, ps):
            gb = gbuf.at[p]
            sb = sbuf.at[ps]
            vbase = c * K

            def edge16(e0, vv, e):
                bidx = jnp.full((LANES, 1), e, jnp.int32)
                vs = lax.gather(
                    vv, bidx, dnums, (1,),
                    mode=lax.GatherScatterMode.PROMISE_IN_BOUNDS)
                sb[e0 + e, :] = gb[e0 + e, :] * vs

            def group_body(g, _):
                e0 = g * LANES
                vv = valblk[bi, pl.ds(vbase + e0, LANES)]
                for e in range(LANES):
                    edge16(e0, vv, e)
                return 0

            lax.fori_loop(0, GFULL, group_body, 0)
            if GREM:
                e0 = GFULL * LANES
                vv = valblk[bi, pl.ds(vbase + e0, LANES)]
                for e in range(GREM):
                    edge16(e0, vv, e)

        def do_block(b, k, first_pred, next_pred):
            LA = NBUF - 1

            def guarded(fn):
                if next_pred is True:
                    fn()
                else:
                    pl.when(next_pred)(fn)

            for c in range(CPB):
                p = c % NBUF
                ps = (2 * b + c) % SBUF
                wait_gather(p)
                pn = (c + LA) % NBUF
                if c < CPB - LA:
                    start_gather(b, c + LA, pn)
                else:
                    nb = 1 - b

                    def nxt(c=c, pn=pn, nb=nb):
                        if c == CPB - LA:
                            wait_idx(nb)
                        start_gather(nb, c - (CPB - LA), pn)
                    guarded(nxt)
                if first_pred is not None and c < SBUF:
                    pl.when(first_pred)(lambda: wait_scatter(ps))
                else:
                    wait_scatter(ps)
                if c == 4:
                    guarded(lambda: start_idx(k + 1, 1 - b))
                compute_chunk(b, c, p, ps)
                start_scatter(b, c, ps)

        start_idx(0, 0)
        wait_idx(0)
        for c in range(NBUF - 1):
            start_gather(0, c, c)

        def blk_pair(i, _):
            do_block(0, 2 * i, i > 0, True)
            do_block(1, 2 * i + 1, None, i < NPAIR - 1)
            return 0

        lax.fori_loop(0, NPAIR, blk_pair, 0)
        for p in range(SBUF):
            wait_scatter(p)

    @pl.when(cid == 0)
    def _():
        edge_loop(x0_hbm)

    @pl.when(cid == 1)
    def _():
        edge_loop(x1_hbm)

    plsc.subcore_barrier()

    @pl.when(cid == 0)
    def _():
        pltpu.sync_copy(acc.at[pl.ds(sid * RPT, RPT)],
                        y0_hbm.at[pl.ds(sid * RPT, RPT)])

    @pl.when(cid == 1)
    def _():
        pltpu.sync_copy(acc.at[pl.ds(sid * RPT, RPT)],
                        y1_hbm.at[pl.ds(sid * RPT, RPT)])


BPT = BATCH // NS
GCH = 128


@functools.partial(
    pl.kernel,
    out_type=(
        jax.ShapeDtypeStruct((N_LAYERS + 2, NC, BATCH, HALF), jnp.float32),
        jax.ShapeDtypeStruct((N_LAYERS + 2, NC, BATCH, HALF), jnp.float32),
    ),
    mesh=_mesh,
    scratch_types=[
        pltpu.VMEM((2, BPT), jnp.int32),
        pltpu.VMEM((2, GCH, HALF), jnp.float32),
        pltpu.VMEM((GCH, HALF), jnp.float32),
        pltpu.SemaphoreType.DMA,
        pltpu.SemaphoreType.DMA,
        pltpu.SemaphoreType.DMA,
    ],
    compiler_params=pltpu.CompilerParams(use_tc_tiling_on_sc=False),
)
def _batch_gather(uid_hbm, iid_hbm,
                  t00, t01, t10, t11, t20, t21, t30, t31,
                  u_out, i_out, idxbuf, gbuf, accbuf, sg0, sg1, sw):
    cid = lax.axis_index("c")
    sid = lax.axis_index("s")
    base0 = sid * BPT
    sem_g = (sg0, sg1)

    def acc_add(p, first):
        gb = gbuf.at[p]

        def body(r, _):
            if first:
                accbuf[r, :] = gb[r, :]
            else:
                accbuf[r, :] = accbuf[r, :] + gb[r, :]
            return 0
        lax.fori_loop(0, GCH, body, 0, unroll=8)

    def acc_scale():
        def body(r, _):
            accbuf[r, :] = accbuf[r, :] * jnp.float32(0.25)
            return 0
        lax.fori_loop(0, GCH, body, 0, unroll=8)

    def wait_gather(p):
        pltpu.make_async_copy(t00.at[pl.ds(0, GCH)], gbuf.at[p],
                              sem_g[p]).wait()

    def run(tabs):
        jobs = []
        for which, (id_hbm, out_hbm, offset) in enumerate(
                ((uid_hbm, u_out, 0), (iid_hbm, i_out, N_USERS))):
            pltpu.sync_copy(id_hbm.at[pl.ds(base0, BPT)], idxbuf.at[which])
            if offset:
                for j in range(BPT // LANES):
                    sl = pl.ds(j * LANES, LANES)
                    idxbuf[which, sl] = idxbuf[which, sl] + jnp.int32(offset)
            for h in range(BPT // GCH):
                jobs.append((which, out_hbm, h))

        stream = [(job, l) for job in jobs for l in range(len(tabs))]

        def idx_ref(job):
            which, _, h = job
            return idxbuf.at[which].at[pl.ds(h * GCH, GCH)]

        job0, l0 = stream[0]
        pltpu.async_copy(tabs[l0].at[idx_ref(job0)], gbuf.at[0], sem_g[0])
        for i, (job, l) in enumerate(stream):
            p = i % 2
            wait_gather(p)
            if i + 1 < len(stream):
                njob, nl = stream[i + 1]
                pltpu.async_copy(tabs[nl].at[idx_ref(njob)],
                                 gbuf.at[1 - p], sem_g[1 - p])
            which, out_hbm, h = job
            base = base0 + h * GCH
            pltpu.async_copy(gbuf.at[p],
                             out_hbm.at[l, cid, pl.ds(base, GCH)], sw)
            acc_add(p, first=(l == 0))
            if l == len(tabs) - 1:
                acc_scale()
                pltpu.sync_copy(accbuf, out_hbm.at[N_LAYERS + 1, cid,
                                                   pl.ds(base, GCH)])
            pltpu.make_async_copy(t00.at[pl.ds(0, GCH)], gbuf.at[p],
                                  sw).wait()

    @pl.when(cid == 0)
    def _():
        run((t00, t10, t20, t30))

    @pl.when(cid == 1)
    def _():
        run((t01, t11, t21, t31))


def kernel(user_id, item_id, adj_row, adj_col, adj_val, user_emb, item_emb):
    user_id = user_id.astype(jnp.int32)
    item_id = item_id.astype(jnp.int32)
    adj_row = adj_row.astype(jnp.int32)
    adj_col = adj_col.astype(jnp.int32)

    pad = jnp.zeros((N_PAD - N, HALF), jnp.float32)
    ego0 = jnp.concatenate([user_emb[:, :HALF], item_emb[:, :HALF], pad],
                           axis=0)
    ego1 = jnp.concatenate([user_emb[:, HALF:], item_emb[:, HALF:], pad],
                           axis=0)

    row2 = adj_row.reshape(CROWS, K)
    col2 = adj_col.reshape(CROWS, K)

    halves = [(ego0, ego1)]
    for _ in range(N_LAYERS):
        x0, x1 = halves[-1]
        halves.append(_spmm(x0, x1, row2, col2, adj_val))

    tabs = [t for pair in halves for t in pair]
    u5, i5 = _batch_gather(user_id, item_id, *tabs)
    u = u5.transpose(0, 2, 1, 3).reshape(N_LAYERS + 2, BATCH, EMB)
    i = i5.transpose(0, 2, 1, 3).reshape(N_LAYERS + 2, BATCH, EMB)
    return (u, i)
